# baseline (speedup 1.0000x reference)
import jax
import jax.numpy as jnp
from jax import lax
from jax.experimental import pallas as pl
from jax.experimental.pallas import tpu as pltpu
from jax.experimental.pallas import tpu_sc as plsc

_N_ROWS = 100_000
_CH = 128
_NSEG = 256
_NEG = float("-inf")

_NW = 32
_RPT = 1120
_SC_START = _N_ROWS - _NW * _RPT
_BLK = 512
_TC_BLOCKS = 126
_TC_ROWS = _TC_BLOCKS * _BLK

_CHUNK = 224
_NCHUNK = _RPT // _CHUNK
_GROUPS = _CHUNK // 16
_CHUNK_EL = _CHUNK * _CH


def _sc_body(feat_hbm, ids_hbm, part_hbm, ids_v, buf, table_v, acc_v,
             sem0, sem1, semi):
    c = lax.axis_index("c")
    s = lax.axis_index("s")
    wid = c * 16 + s
    base = pl.multiple_of(_SC_START + wid * _RPT, 16)

    idcp = pltpu.async_copy(ids_hbm.at[pl.ds(base, _RPT)], ids_v, semi)

    neg = jnp.full((16,), _NEG, jnp.float32)
    neg2 = neg.reshape(1, 16)

    @pl.loop(0, _NSEG)
    def _(i):
        for k in range(8):
            table_v[pl.ds(i, 1), pl.ds(k * 16, 16)] = neg2

    pltpu.async_copy(feat_hbm.at[pl.ds(base * _CH, _CHUNK_EL)],
                     buf.at[pl.ds(0, _CHUNK_EL)], sem0)
    pltpu.async_copy(feat_hbm.at[pl.ds((base + _CHUNK) * _CH, _CHUNK_EL)],
                     buf.at[pl.ds(_CHUNK_EL, _CHUNK_EL)], sem1)

    for k in range(8):
        acc_v[pl.ds(k * 16, 16)] = neg

    idcp.wait()
    cur0 = ids_v[pl.ds(0, 16)][0]

    def flush(seg, acc):
        for k in range(8):
            table_v[pl.ds(seg, 1), pl.ds(k * 16, 16)] = acc[k].reshape(1, 16)

    def chunk_step(j, cur):
        even = lax.rem(j, 2) == 0

        @pl.when(even)
        def _():
            pltpu.make_async_copy(feat_hbm.at[pl.ds(0, _CHUNK_EL)],
                                  buf.at[pl.ds(0, _CHUNK_EL)], sem0).wait()

        @pl.when(jnp.logical_not(even))
        def _():
            pltpu.make_async_copy(feat_hbm.at[pl.ds(0, _CHUNK_EL)],
                                  buf.at[pl.ds(_CHUNK_EL, _CHUNK_EL)],
                                  sem1).wait()

        par = lax.rem(j, 2) * _CHUNK_EL

        def group_step(g, cur):
            idv = ids_v[pl.ds(j * _CHUNK + g * 16, 16)]
            rbase = par + g * 16 * _CH

            def fast(cur):
                acc = [acc_v[pl.ds(k * 16, 16)] for k in range(8)]
                for r in range(16):
                    rb = rbase + r * _CH
                    for k in range(8):
                        x = buf[pl.ds(rb + k * 16, 16)]
                        acc[k] = jnp.maximum(acc[k], x)
                for k in range(8):
                    acc_v[pl.ds(k * 16, 16)] = acc[k]
                return cur

            def slow(cur):
                acc = [acc_v[pl.ds(k * 16, 16)] for k in range(8)]
                for r in range(16):
                    idr = idv[r]
                    fresh = idr != cur
                    cur_old = cur
                    acc_old = list(acc)

                    @pl.when(fresh)
                    def _():
                        flush(cur_old, acc_old)

                    rb = rbase + r * _CH
                    for k in range(8):
                        x = buf[pl.ds(rb + k * 16, 16)]
                        acc[k] = jnp.where(fresh, x,
                                           jnp.maximum(acc[k], x))
                    cur = jnp.where(fresh, idr, cur)
                for k in range(8):
                    acc_v[pl.ds(k * 16, 16)] = acc[k]
                return cur

            return lax.cond(idv[15] == cur, fast, slow, cur)

        cur = lax.fori_loop(0, _GROUPS, group_step, cur)

        nxt = (base + (j + 2) * _CHUNK) * _CH

        @pl.when(even & (j + 2 < _NCHUNK))
        def _():
            pltpu.async_copy(feat_hbm.at[pl.ds(nxt, _CHUNK_EL)],
                             buf.at[pl.ds(0, _CHUNK_EL)], sem0)

        @pl.when(jnp.logical_not(even) & (j + 2 < _NCHUNK))
        def _():
            pltpu.async_copy(feat_hbm.at[pl.ds(nxt, _CHUNK_EL)],
                             buf.at[pl.ds(_CHUNK_EL, _CHUNK_EL)], sem1)

        return cur

    cur = lax.fori_loop(0, _NCHUNK, chunk_step, cur0)

    flush(cur, [acc_v[pl.ds(k * 16, 16)] for k in range(8)])
    pltpu.sync_copy(table_v, part_hbm.at[pl.ds(wid * _NSEG, _NSEG), :])


_sc_segmax = pl.kernel(
    _sc_body,
    out_type=jax.ShapeDtypeStruct((_NW * _NSEG, _CH), jnp.float32),
    mesh=plsc.VectorSubcoreMesh(core_axis_name="c", subcore_axis_name="s"),
    scratch_types=[
        pltpu.VMEM((_RPT,), jnp.int32),
        pltpu.VMEM((2 * _CHUNK_EL,), jnp.float32),
        pltpu.VMEM((_NSEG, _CH), jnp.float32),
        pltpu.VMEM((_CH,), jnp.float32),
        pltpu.SemaphoreType.DMA,
        pltpu.SemaphoreType.DMA,
        pltpu.SemaphoreType.DMA,
    ],
)


def _tc_body(starts_ref, blo_ref, bhi_ref, x_ref, o_ref):
    i = pl.program_id(0)

    @pl.when(i == 0)
    def _():
        o_ref[...] = jnp.full((_NSEG, _CH), _NEG, jnp.float32)

    x = x_ref[...]
    lo = blo_ref[i]
    hi = bhi_ref[i]
    base_row = i * _BLK
    riota = lax.broadcasted_iota(jnp.int32, (_BLK, _CH), 0)

    def seg_step(k, _):
        sid = lo + k
        a = starts_ref[sid] - base_row
        b = starts_ref[sid + 1] - base_row
        m = (riota >= a) & (riota < b)
        mk = jnp.max(jnp.where(m, x, _NEG), axis=0).reshape(1, _CH)
        o_ref[pl.ds(sid, 1), :] = jnp.maximum(o_ref[pl.ds(sid, 1), :], mk)
        return 0

    lax.fori_loop(0, hi - lo + 1, seg_step, 0)


_tc_segmax = pl.pallas_call(
    _tc_body,
    grid_spec=pltpu.PrefetchScalarGridSpec(
        num_scalar_prefetch=3,
        grid=(_TC_BLOCKS,),
        in_specs=[
            pl.BlockSpec((_BLK, _CH), lambda i, *_: (i, 0)),
        ],
        out_specs=pl.BlockSpec((_NSEG, _CH), lambda i, *_: (0, 0)),
    ),
    out_shape=jax.ShapeDtypeStruct((_NSEG, _CH), jnp.float32),
)


def _combine_body(p_ref, t_ref, o_ref):
    acc = t_ref[...]
    for t in range(_NW):
        acc = jnp.maximum(acc, p_ref[pl.ds(t * _NSEG, _NSEG), :])
    o_ref[...] = acc


_combine = pl.pallas_call(
    _combine_body,
    out_shape=jax.ShapeDtypeStruct((_NSEG, _CH), jnp.float32),
)


@jax.jit
def kernel(features, segment_ids):
    feat2d = features.reshape(_N_ROWS, _CH)
    feat = features.reshape(_N_ROWS * _CH)
    ids = segment_ids.astype(jnp.int32)

    starts = jnp.searchsorted(
        ids, jnp.arange(_NSEG + 5, dtype=jnp.int32)).astype(jnp.int32)
    blo = ids[:_TC_ROWS:_BLK]
    bhi = ids[_BLK - 1:_TC_ROWS:_BLK]

    part_sc = _sc_segmax(feat, ids)
    part_tc = _tc_segmax(starts, blo, bhi, feat2d)
    return _combine(part_sc, part_tc)


# speedup vs baseline: 1.7394x; 1.7394x over previous
import jax
import jax.numpy as jnp
from jax import lax
from jax.experimental import pallas as pl
from jax.experimental.pallas import tpu as pltpu
from jax.experimental.pallas import tpu_sc as plsc

_N_ROWS = 100_000
_CH = 128
_NSEG = 256
_NEG = float("-inf")

_NW = 32
_RPT = 2016
_SC_START = _N_ROWS - _NW * _RPT
_BLK = 512
_TC_BLOCKS = 70
_TC_ROWS = _TC_BLOCKS * _BLK

_CHUNK = 224
_NCHUNK = _RPT // _CHUNK
_GROUPS = _CHUNK // 16
_CHUNK_EL = _CHUNK * _CH


def _sc_body(feat_hbm, ids_hbm, part_hbm, ids_v, buf, table_v, acc_v,
             sem0, sem1, semi):
    c = lax.axis_index("c")
    s = lax.axis_index("s")
    wid = c * 16 + s
    base = pl.multiple_of(_SC_START + wid * _RPT, 16)

    idcp = pltpu.async_copy(ids_hbm.at[pl.ds(base, _RPT)], ids_v, semi)

    neg = jnp.full((16,), _NEG, jnp.float32)
    neg2 = neg.reshape(1, 16)

    @pl.loop(0, _NSEG)
    def _(i):
        for k in range(8):
            table_v[pl.ds(i, 1), pl.ds(k * 16, 16)] = neg2

    pltpu.async_copy(feat_hbm.at[pl.ds(base * _CH, _CHUNK_EL)],
                     buf.at[pl.ds(0, _CHUNK_EL)], sem0)
    pltpu.async_copy(feat_hbm.at[pl.ds((base + _CHUNK) * _CH, _CHUNK_EL)],
                     buf.at[pl.ds(_CHUNK_EL, _CHUNK_EL)], sem1)

    for k in range(8):
        acc_v[pl.ds(k * 16, 16)] = neg

    idcp.wait()
    cur0 = ids_v[pl.ds(0, 16)][0]

    def flush(seg, acc):
        for k in range(8):
            table_v[pl.ds(seg, 1), pl.ds(k * 16, 16)] = acc[k].reshape(1, 16)

    def chunk_step(j, cur):
        even = lax.rem(j, 2) == 0

        @pl.when(even)
        def _():
            pltpu.make_async_copy(feat_hbm.at[pl.ds(0, _CHUNK_EL)],
                                  buf.at[pl.ds(0, _CHUNK_EL)], sem0).wait()

        @pl.when(jnp.logical_not(even))
        def _():
            pltpu.make_async_copy(feat_hbm.at[pl.ds(0, _CHUNK_EL)],
                                  buf.at[pl.ds(_CHUNK_EL, _CHUNK_EL)],
                                  sem1).wait()

        par = lax.rem(j, 2) * _CHUNK_EL

        def group_step(g, cur):
            idv = ids_v[pl.ds(j * _CHUNK + g * 16, 16)]
            rbase = par + g * 16 * _CH

            def fast(cur):
                acc = [acc_v[pl.ds(k * 16, 16)] for k in range(8)]
                for r in range(16):
                    rb = rbase + r * _CH
                    for k in range(8):
                        x = buf[pl.ds(rb + k * 16, 16)]
                        acc[k] = jnp.maximum(acc[k], x)
                for k in range(8):
                    acc_v[pl.ds(k * 16, 16)] = acc[k]
                return cur

            def slow(cur):
                acc = [acc_v[pl.ds(k * 16, 16)] for k in range(8)]
                for r in range(16):
                    idr = idv[r]
                    fresh = idr != cur
                    cur_old = cur
                    acc_old = list(acc)

                    @pl.when(fresh)
                    def _():
                        flush(cur_old, acc_old)

                    rb = rbase + r * _CH
                    for k in range(8):
                        x = buf[pl.ds(rb + k * 16, 16)]
                        acc[k] = jnp.where(fresh, x,
                                           jnp.maximum(acc[k], x))
                    cur = jnp.where(fresh, idr, cur)
                for k in range(8):
                    acc_v[pl.ds(k * 16, 16)] = acc[k]
                return cur

            return lax.cond(idv[15] == cur, fast, slow, cur)

        cur = lax.fori_loop(0, _GROUPS, group_step, cur)

        nxt = (base + (j + 2) * _CHUNK) * _CH

        @pl.when(even & (j + 2 < _NCHUNK))
        def _():
            pltpu.async_copy(feat_hbm.at[pl.ds(nxt, _CHUNK_EL)],
                             buf.at[pl.ds(0, _CHUNK_EL)], sem0)

        @pl.when(jnp.logical_not(even) & (j + 2 < _NCHUNK))
        def _():
            pltpu.async_copy(feat_hbm.at[pl.ds(nxt, _CHUNK_EL)],
                             buf.at[pl.ds(_CHUNK_EL, _CHUNK_EL)], sem1)

        return cur

    cur = lax.fori_loop(0, _NCHUNK, chunk_step, cur0)

    flush(cur, [acc_v[pl.ds(k * 16, 16)] for k in range(8)])
    pltpu.sync_copy(table_v, part_hbm.at[pl.ds(wid * _NSEG, _NSEG), :])


_sc_segmax = pl.kernel(
    _sc_body,
    out_type=jax.ShapeDtypeStruct((_NW * _NSEG, _CH), jnp.float32),
    mesh=plsc.VectorSubcoreMesh(core_axis_name="c", subcore_axis_name="s"),
    scratch_types=[
        pltpu.VMEM((_RPT,), jnp.int32),
        pltpu.VMEM((2 * _CHUNK_EL,), jnp.float32),
        pltpu.VMEM((_NSEG, _CH), jnp.float32),
        pltpu.VMEM((_CH,), jnp.float32),
        pltpu.SemaphoreType.DMA,
        pltpu.SemaphoreType.DMA,
        pltpu.SemaphoreType.DMA,
    ],
)


def _tc_body(ids_ref, x_ref, o_ref):
    i = pl.program_id(0)

    @pl.when(i == 0)
    def _():
        o_ref[...] = jnp.full((_NSEG, _CH), _NEG, jnp.float32)

    x = x_ref[...]
    ids_blk = ids_ref[...]
    lo = ids_blk[0, 0, 0]
    hi = ids_blk[0, _BLK // _CH - 1, _CH - 1]
    riota = lax.broadcasted_iota(jnp.uint32, (_BLK, _CH), 0)

    def seg_step(k, a):
        sid = lo + k
        b = jnp.sum((ids_blk <= sid).astype(jnp.int32))
        m = (riota - a.astype(jnp.uint32)) < (b - a).astype(jnp.uint32)
        mk = jnp.max(jnp.where(m, x, _NEG), axis=0).reshape(1, _CH)
        o_ref[pl.ds(sid, 1), :] = jnp.maximum(o_ref[pl.ds(sid, 1), :], mk)
        return b

    lax.fori_loop(0, hi - lo + 1, seg_step, jnp.int32(0))


_tc_segmax = pl.pallas_call(
    _tc_body,
    grid=(_TC_BLOCKS,),
    in_specs=[
        pl.BlockSpec((1, _BLK // _CH, _CH), lambda i: (i, 0, 0)),
        pl.BlockSpec((_BLK, _CH), lambda i: (i, 0)),
    ],
    out_specs=pl.BlockSpec((_NSEG, _CH), lambda i: (0, 0)),
    out_shape=jax.ShapeDtypeStruct((_NSEG, _CH), jnp.float32),
)


def _combine_body(p_ref, t_ref, o_ref):
    acc = t_ref[...]
    for t in range(_NW):
        acc = jnp.maximum(acc, p_ref[pl.ds(t * _NSEG, _NSEG), :])
    o_ref[...] = acc


_combine = pl.pallas_call(
    _combine_body,
    out_shape=jax.ShapeDtypeStruct((_NSEG, _CH), jnp.float32),
)


@jax.jit
def kernel(features, segment_ids):
    feat2d = features.reshape(_N_ROWS, _CH)
    feat = features.reshape(_N_ROWS * _CH)
    ids = segment_ids.astype(jnp.int32)

    ids3 = ids[:_TC_ROWS].reshape(_TC_BLOCKS, _BLK // _CH, _CH)

    part_sc = _sc_segmax(feat, ids)
    part_tc = _tc_segmax(ids3, feat2d)
    return _combine(part_sc, part_tc)


# speedup vs baseline: 1.7657x; 1.0151x over previous
import jax
import jax.numpy as jnp
from jax import lax
from jax.experimental import pallas as pl
from jax.experimental.pallas import tpu as pltpu
from jax.experimental.pallas import tpu_sc as plsc

_N_ROWS = 100_000
_CH = 128
_NSEG = 256
_NEG = float("-inf")

_NW = 32
_RPT = 2016
_SC_START = _N_ROWS - _NW * _RPT
_BLK = 512
_TC_BLOCKS = 70
_TC_ROWS = _TC_BLOCKS * _BLK

_CHUNK = 224
_NCHUNK = _RPT // _CHUNK
_GROUPS = _CHUNK // 16
_CHUNK_EL = _CHUNK * _CH


def _sc_body(feat_hbm, ids_hbm, part_hbm, ids_v, buf, table_v, acc_v,
             sem0, sem1, semi):
    c = lax.axis_index("c")
    s = lax.axis_index("s")
    wid = c * 16 + s
    base = pl.multiple_of(_SC_START + wid * _RPT, 16)

    idcp = pltpu.async_copy(ids_hbm.at[pl.ds(base, _RPT)], ids_v, semi)

    neg = jnp.full((16,), _NEG, jnp.float32)
    neg2 = neg.reshape(1, 16)

    @pl.loop(0, _NSEG)
    def _(i):
        for k in range(8):
            table_v[pl.ds(i, 1), pl.ds(k * 16, 16)] = neg2

    pltpu.async_copy(feat_hbm.at[pl.ds(base * _CH, _CHUNK_EL)],
                     buf.at[pl.ds(0, _CHUNK_EL)], sem0)
    pltpu.async_copy(feat_hbm.at[pl.ds((base + _CHUNK) * _CH, _CHUNK_EL)],
                     buf.at[pl.ds(_CHUNK_EL, _CHUNK_EL)], sem1)

    for k in range(8):
        acc_v[pl.ds(k * 16, 16)] = neg

    idcp.wait()
    cur0 = ids_v[pl.ds(0, 16)][0]

    def flush(seg, acc):
        for k in range(8):
            table_v[pl.ds(seg, 1), pl.ds(k * 16, 16)] = acc[k].reshape(1, 16)

    def chunk_step(j, cur):
        even = lax.rem(j, 2) == 0

        @pl.when(even)
        def _():
            pltpu.make_async_copy(feat_hbm.at[pl.ds(0, _CHUNK_EL)],
                                  buf.at[pl.ds(0, _CHUNK_EL)], sem0).wait()

        @pl.when(jnp.logical_not(even))
        def _():
            pltpu.make_async_copy(feat_hbm.at[pl.ds(0, _CHUNK_EL)],
                                  buf.at[pl.ds(_CHUNK_EL, _CHUNK_EL)],
                                  sem1).wait()

        par = lax.rem(j, 2) * _CHUNK_EL

        def group_step(g, cur):
            idv = ids_v[pl.ds(j * _CHUNK + g * 16, 16)]
            rbase = par + g * 16 * _CH

            def fast(cur):
                acc = [acc_v[pl.ds(k * 16, 16)] for k in range(8)]
                for r in range(16):
                    rb = rbase + r * _CH
                    for k in range(8):
                        x = buf[pl.ds(rb + k * 16, 16)]
                        acc[k] = jnp.maximum(acc[k], x)
                for k in range(8):
                    acc_v[pl.ds(k * 16, 16)] = acc[k]
                return cur

            def slow(cur):
                acc = [acc_v[pl.ds(k * 16, 16)] for k in range(8)]
                for r in range(16):
                    idr = idv[r]
                    fresh = idr != cur
                    cur_old = cur
                    acc_old = list(acc)

                    @pl.when(fresh)
                    def _():
                        flush(cur_old, acc_old)

                    rb = rbase + r * _CH
                    for k in range(8):
                        x = buf[pl.ds(rb + k * 16, 16)]
                        acc[k] = jnp.where(fresh, x,
                                           jnp.maximum(acc[k], x))
                    cur = jnp.where(fresh, idr, cur)
                for k in range(8):
                    acc_v[pl.ds(k * 16, 16)] = acc[k]
                return cur

            return lax.cond(idv[15] == cur, fast, slow, cur)

        cur = lax.fori_loop(0, _GROUPS, group_step, cur)

        nxt = (base + (j + 2) * _CHUNK) * _CH

        @pl.when(even & (j + 2 < _NCHUNK))
        def _():
            pltpu.async_copy(feat_hbm.at[pl.ds(nxt, _CHUNK_EL)],
                             buf.at[pl.ds(0, _CHUNK_EL)], sem0)

        @pl.when(jnp.logical_not(even) & (j + 2 < _NCHUNK))
        def _():
            pltpu.async_copy(feat_hbm.at[pl.ds(nxt, _CHUNK_EL)],
                             buf.at[pl.ds(_CHUNK_EL, _CHUNK_EL)], sem1)

        return cur

    cur = lax.fori_loop(0, _NCHUNK, chunk_step, cur0)

    flush(cur, [acc_v[pl.ds(k * 16, 16)] for k in range(8)])
    pltpu.sync_copy(table_v, part_hbm.at[pl.ds(wid * _NSEG, _NSEG), :])


_sc_segmax = pl.kernel(
    _sc_body,
    out_type=jax.ShapeDtypeStruct((_NW * _NSEG, _CH), jnp.float32),
    mesh=plsc.VectorSubcoreMesh(core_axis_name="c", subcore_axis_name="s"),
    scratch_types=[
        pltpu.VMEM((_RPT,), jnp.int32),
        pltpu.VMEM((2 * _CHUNK_EL,), jnp.float32),
        pltpu.VMEM((_NSEG, _CH), jnp.float32),
        pltpu.VMEM((_CH,), jnp.float32),
        pltpu.SemaphoreType.DMA,
        pltpu.SemaphoreType.DMA,
        pltpu.SemaphoreType.DMA,
    ],
)


def _tc_body(ids_ref, x_ref, o_ref):
    i = pl.program_id(0)

    @pl.when(i == 0)
    def _():
        o_ref[...] = jnp.full((_NSEG, _CH), _NEG, jnp.float32)

    x = x_ref[...]
    ids_blk = ids_ref[...]
    lo = ids_blk[0, 0, 0]
    hi = ids_blk[0, _BLK // _CH - 1, _CH - 1]
    riota = lax.broadcasted_iota(jnp.int32, (_BLK, _CH), 0)

    def seg_step(k, a):
        sid = lo + k
        b = jnp.sum((ids_blk <= sid).astype(jnp.int32),
                    axis=(0, 1, 2), keepdims=True).reshape(1, 1)
        m = (riota >= a) & (riota < b)
        mk = jnp.max(jnp.where(m, x, _NEG), axis=0, keepdims=True)
        o_ref[pl.ds(sid, 1), :] = jnp.maximum(o_ref[pl.ds(sid, 1), :], mk)
        return b

    lax.fori_loop(0, hi - lo + 1, seg_step, jnp.zeros((1, 1), jnp.int32))


_tc_segmax = pl.pallas_call(
    _tc_body,
    grid=(_TC_BLOCKS,),
    in_specs=[
        pl.BlockSpec((1, _BLK // _CH, _CH), lambda i: (i, 0, 0)),
        pl.BlockSpec((_BLK, _CH), lambda i: (i, 0)),
    ],
    out_specs=pl.BlockSpec((_NSEG, _CH), lambda i: (0, 0)),
    out_shape=jax.ShapeDtypeStruct((_NSEG, _CH), jnp.float32),
)


def _combine_body(p_ref, t_ref, o_ref):
    acc = t_ref[...]
    for t in range(_NW):
        acc = jnp.maximum(acc, p_ref[pl.ds(t * _NSEG, _NSEG), :])
    o_ref[...] = acc


_combine = pl.pallas_call(
    _combine_body,
    out_shape=jax.ShapeDtypeStruct((_NSEG, _CH), jnp.float32),
)


@jax.jit
def kernel(features, segment_ids):
    feat2d = features.reshape(_N_ROWS, _CH)
    feat = features.reshape(_N_ROWS * _CH)
    ids = segment_ids.astype(jnp.int32)

    ids3 = ids[:_TC_ROWS].reshape(_TC_BLOCKS, _BLK // _CH, _CH)

    part_sc = _sc_segmax(feat, ids)
    part_tc = _tc_segmax(ids3, feat2d)
    return _combine(part_sc, part_tc)


# speedup vs baseline: 2.4991x; 1.4153x over previous
import jax
import jax.numpy as jnp
from jax import lax
from jax.experimental import pallas as pl
from jax.experimental.pallas import tpu as pltpu
from jax.experimental.pallas import tpu_sc as plsc

_N_ROWS = 100_000
_CH = 128
_NSEG = 256
_NW = 32
_RPT = 3136
_LAST_BASE = _N_ROWS - _RPT
_CHUNK = 224
_NCHUNK = _RPT // _CHUNK
_GROUPS = _CHUNK // 16
_CHUNK_EL = _CHUNK * _CH
_TBL = _NSEG * _CH
_NEG = float("-inf")


def _sc_body(feat_hbm, ids_hbm, part_hbm, ids_v, buf, table_v, acc_v,
             sem0, sem1, semi):
    c = lax.axis_index("c")
    s = lax.axis_index("s")
    wid = c * 16 + s
    base = pl.multiple_of(jnp.minimum(wid * _RPT, _LAST_BASE), 16)

    idcp = pltpu.async_copy(ids_hbm.at[pl.ds(base, _RPT)], ids_v, semi)

    pltpu.async_copy(feat_hbm.at[pl.ds(base * _CH, _CHUNK_EL)],
                     buf.at[pl.ds(0, _CHUNK_EL)], sem0)
    pltpu.async_copy(feat_hbm.at[pl.ds((base + _CHUNK) * _CH, _CHUNK_EL)],
                     buf.at[pl.ds(_CHUNK_EL, _CHUNK_EL)], sem1)

    neg = jnp.full((16,), _NEG, jnp.float32)
    neg2 = neg.reshape(1, 16)

    @pl.loop(0, _NSEG)
    def _(i):
        for k in range(8):
            table_v[pl.ds(i, 1), pl.ds(k * 16, 16)] = neg2

    for k in range(8):
        acc_v[pl.ds(k * 16, 16)] = neg

    idcp.wait()
    cur0 = ids_v[pl.ds(0, 16)][0]

    def flush(seg, acc):
        for k in range(8):
            table_v[pl.ds(seg, 1), pl.ds(k * 16, 16)] = acc[k].reshape(1, 16)

    def chunk_step(j, cur):
        even = lax.rem(j, 2) == 0

        @pl.when(even)
        def _():
            pltpu.make_async_copy(feat_hbm.at[pl.ds(0, _CHUNK_EL)],
                                  buf.at[pl.ds(0, _CHUNK_EL)], sem0).wait()

        @pl.when(jnp.logical_not(even))
        def _():
            pltpu.make_async_copy(feat_hbm.at[pl.ds(0, _CHUNK_EL)],
                                  buf.at[pl.ds(_CHUNK_EL, _CHUNK_EL)],
                                  sem1).wait()

        par = lax.rem(j, 2) * _CHUNK_EL

        def group_step(g, cur):
            idv = ids_v[pl.ds(j * _CHUNK + g * 16, 16)]
            rbase = par + g * 16 * _CH

            def fast(cur):
                acc = [acc_v[pl.ds(k * 16, 16)] for k in range(8)]
                for r in range(16):
                    rb = rbase + r * _CH
                    for k in range(8):
                        x = buf[pl.ds(rb + k * 16, 16)]
                        acc[k] = jnp.maximum(acc[k], x)
                for k in range(8):
                    acc_v[pl.ds(k * 16, 16)] = acc[k]
                return cur

            def slow(cur):
                acc = [acc_v[pl.ds(k * 16, 16)] for k in range(8)]
                for r in range(16):
                    idr = idv[r]
                    fresh = idr != cur
                    cur_old = cur
                    acc_old = list(acc)

                    @pl.when(fresh)
                    def _():
                        flush(cur_old, acc_old)

                    rb = rbase + r * _CH
                    for k in range(8):
                        x = buf[pl.ds(rb + k * 16, 16)]
                        acc[k] = jnp.where(fresh, x,
                                           jnp.maximum(acc[k], x))
                    cur = jnp.where(fresh, idr, cur)
                for k in range(8):
                    acc_v[pl.ds(k * 16, 16)] = acc[k]
                return cur

            return lax.cond(idv[15] == cur, fast, slow, cur)

        cur = lax.fori_loop(0, _GROUPS, group_step, cur)

        nxt = (base + (j + 2) * _CHUNK) * _CH

        @pl.when(even & (j + 2 < _NCHUNK))
        def _():
            pltpu.async_copy(feat_hbm.at[pl.ds(nxt, _CHUNK_EL)],
                             buf.at[pl.ds(0, _CHUNK_EL)], sem0)

        @pl.when(jnp.logical_not(even) & (j + 2 < _NCHUNK))
        def _():
            pltpu.async_copy(feat_hbm.at[pl.ds(nxt, _CHUNK_EL)],
                             buf.at[pl.ds(_CHUNK_EL, _CHUNK_EL)], sem1)

        return cur

    cur = lax.fori_loop(0, _NCHUNK, chunk_step, cur0)

    flush(cur, [acc_v[pl.ds(k * 16, 16)] for k in range(8)])
    pltpu.sync_copy(table_v, part_hbm.at[pl.ds(wid * _NSEG, _NSEG), :])


_sc_segmax = pl.kernel(
    _sc_body,
    out_type=jax.ShapeDtypeStruct((_NW * _NSEG, _CH), jnp.float32),
    mesh=plsc.VectorSubcoreMesh(core_axis_name="c", subcore_axis_name="s"),
    scratch_types=[
        pltpu.VMEM((_RPT,), jnp.int32),
        pltpu.VMEM((2 * _CHUNK_EL,), jnp.float32),
        pltpu.VMEM((_NSEG, _CH), jnp.float32),
        pltpu.VMEM((_CH,), jnp.float32),
        pltpu.SemaphoreType.DMA,
        pltpu.SemaphoreType.DMA,
        pltpu.SemaphoreType.DMA,
    ],
)


def _combine_body(p_ref, o_ref):
    acc = p_ref[pl.ds(0, _NSEG), :]
    for t in range(1, _NW):
        acc = jnp.maximum(acc, p_ref[pl.ds(t * _NSEG, _NSEG), :])
    o_ref[...] = acc


_combine = pl.pallas_call(
    _combine_body,
    out_shape=jax.ShapeDtypeStruct((_NSEG, _CH), jnp.float32),
)


@jax.jit
def kernel(features, segment_ids):
    feat = features.reshape(_N_ROWS * _CH)
    ids = segment_ids.astype(jnp.int32)
    part = _sc_segmax(feat, ids)
    return _combine(part)
